# baseline (speedup 1.0000x reference)
import functools

import jax
import jax.numpy as jnp
from jax import lax
from jax.experimental import pallas as pl
from jax.experimental.pallas import tpu as pltpu
from jax.experimental.pallas import tpu_sc as plsc

_DROP_PROB = 0.2
_MIN_KEEP = 1
_EPS = 1e-8


def _keep_flags(imp_ref, noise_ref, n_routes, n_keep):
    s = [1.0 / (imp_ref[i] + _EPS) + noise_ref[i] for i in range(n_routes)]
    keeps = []
    for i in range(n_routes):
        rank = jnp.int32(0)
        for j in range(n_routes):
            if j < i:
                rank += (s[j] <= s[i]).astype(jnp.int32)
            elif j > i:
                rank += (s[j] < s[i]).astype(jnp.int32)
        keeps.append(rank < n_keep)
    return keeps


def _body(imp_ref, noise_ref, x_hbm, o_ref, km_ref, buf, sem, kf_ref, *,
          n_routes, n_keep, scale, block_t, nb):
    b = pl.program_id(0)

    n_slots = buf.shape[0]

    @pl.when(b == 0)
    def _():
        flags = _keep_flags(imp_ref, noise_ref, n_routes, n_keep)
        for i in range(n_routes):
            kf_ref[i] = flags[i].astype(jnp.int32)
            km_ref[i] = flags[i].astype(jnp.float32)
        for r in range(n_routes):
            @pl.when(jnp.logical_not(flags[r]))
            def _():
                for s in range(n_slots):
                    buf[s, :, r, :] = jnp.zeros((block_t, buf.shape[3]),
                                                jnp.float32)

    keeps = [kf_ref[i] != 0 for i in range(n_routes)]

    def issue(block_idx, slot):
        for r in range(n_routes):
            @pl.when(keeps[r])
            def _():
                pltpu.make_async_copy(
                    x_hbm.at[pl.ds(block_idx * block_t, block_t), r, :],
                    buf.at[slot, :, r, :],
                    sem.at[slot]).start()

    @pl.when(b == 0)
    def _():
        for i in range(n_slots - 1):
            issue(i, i)

    @pl.when(b + n_slots - 1 < nb)
    def _():
        issue(b + n_slots - 1, (b + n_slots - 1) % n_slots)

    cur = b % n_slots
    for r in range(n_routes):
        @pl.when(keeps[r])
        def _():
            pltpu.make_async_copy(
                x_hbm.at[pl.ds(0, block_t), r, :],
                buf.at[cur, :, r, :],
                sem.at[cur]).wait()
    o_ref[...] = buf[cur] * jnp.float32(scale)


def _sc_mask_body(n_routes, n_keep, imp_hbm, noise_hbm, km_hbm,
                  imp_v, noise_v, score_v, km_v):
    c = lax.axis_index("c")
    s = lax.axis_index("s")

    @pl.when(jnp.logical_and(c == 0, s == 0))
    def _():
        pltpu.sync_copy(imp_hbm, imp_v)
        pltpu.sync_copy(noise_hbm, noise_v)
        score = 1.0 / (imp_v[...] + _EPS) + noise_v[...]
        score_v[...] = score
        iota = lax.iota(jnp.int32, n_routes)
        rank = jnp.zeros((n_routes,), jnp.int32)
        neg_inf = jnp.float32(-3.4e38)
        for j in range(n_routes):
            sj_scalar = lax.reduce_max(
                jnp.where(iota == j, score, neg_inf), (0,))
            sj = lax.broadcast_in_dim(sj_scalar, (n_routes,), ())
            beats = jnp.logical_or(
                sj < score,
                jnp.logical_and(sj == score,
                                jnp.full((n_routes,), j, jnp.int32) < iota))
            rank = rank + jnp.where(beats, jnp.int32(1), jnp.int32(0))
        km_v[...] = jnp.where(rank < n_keep, jnp.float32(1.0),
                              jnp.float32(0.0))
        pltpu.sync_copy(km_v, km_hbm)


def _sc_keep_mask(importance, noise, n_routes, n_keep):
    mesh = plsc.VectorSubcoreMesh(core_axis_name="c", subcore_axis_name="s")
    kern = functools.partial(
        pl.kernel,
        mesh=mesh,
        out_type=jax.ShapeDtypeStruct((n_routes,), jnp.float32),
        scratch_types=[pltpu.VMEM((n_routes,), jnp.float32),
                       pltpu.VMEM((n_routes,), jnp.float32),
                       pltpu.VMEM((n_routes,), jnp.float32),
                       pltpu.VMEM((n_routes,), jnp.float32)],
        compiler_params=pltpu.CompilerParams(needs_layout_passes=False),
    )(functools.partial(_sc_mask_body, n_routes, n_keep))
    return kern(importance, noise)


def kernel(x, importance):
    n_tokens, n_routes, d = x.shape
    n_keep = max(_MIN_KEEP, int(n_routes * (1.0 - _DROP_PROB)))
    scale = n_routes / float(n_keep)
    noise = jax.random.uniform(jax.random.key(42), (n_routes,),
                               importance.dtype) * 0.5

    block_t = 64
    nb = n_tokens // block_t
    body = functools.partial(_body, n_routes=n_routes, n_keep=n_keep,
                             scale=scale, block_t=block_t, nb=nb)
    out, km = pl.pallas_call(
        body,
        grid=(nb,),
        in_specs=[pl.BlockSpec(memory_space=pltpu.SMEM),
                  pl.BlockSpec(memory_space=pltpu.SMEM),
                  pl.BlockSpec(memory_space=pl.ANY)],
        out_specs=[pl.BlockSpec((block_t, n_routes, d), lambda b: (b, 0, 0)),
                   pl.BlockSpec(memory_space=pltpu.SMEM)],
        out_shape=[jax.ShapeDtypeStruct((n_tokens, n_routes, d), jnp.float32),
                   jax.ShapeDtypeStruct((n_routes,), jnp.float32)],
        scratch_shapes=[pltpu.VMEM((6, block_t, n_routes, d), jnp.float32),
                        pltpu.SemaphoreType.DMA((6,)),
                        pltpu.SMEM((n_routes,), jnp.int32)],
        compiler_params=pltpu.CompilerParams(
            dimension_semantics=("arbitrary",)),
    )(importance, noise, x)
    return out, km


# speedup vs baseline: 1.0232x; 1.0232x over previous
import functools

import jax
import jax.numpy as jnp
from jax import lax
from jax.experimental import pallas as pl
from jax.experimental.pallas import tpu as pltpu
from jax.experimental.pallas import tpu_sc as plsc

_DROP_PROB = 0.2
_MIN_KEEP = 1
_EPS = 1e-8


def _keep_flags(imp_ref, noise_ref, n_routes, n_keep):
    s = [1.0 / (imp_ref[i] + _EPS) + noise_ref[i] for i in range(n_routes)]
    keeps = []
    for i in range(n_routes):
        rank = jnp.int32(0)
        for j in range(n_routes):
            if j < i:
                rank += (s[j] <= s[i]).astype(jnp.int32)
            elif j > i:
                rank += (s[j] < s[i]).astype(jnp.int32)
        keeps.append(rank < n_keep)
    return keeps


def _body(imp_ref, noise_ref, x_hbm, o_ref, km_ref, buf, sem, kf_ref, *,
          n_routes, n_keep, scale, block_t, nb):
    b = pl.program_id(0)

    n_slots = buf.shape[0]

    @pl.when(b == 0)
    def _():
        flags = _keep_flags(imp_ref, noise_ref, n_routes, n_keep)
        for i in range(n_routes):
            kf_ref[i] = flags[i].astype(jnp.int32)
            km_ref[i] = flags[i].astype(jnp.float32)

    keeps = [kf_ref[i] != 0 for i in range(n_routes)]

    def issue(block_idx, slot):
        for r in range(n_routes):
            @pl.when(keeps[r])
            def _():
                pltpu.make_async_copy(
                    x_hbm.at[pl.ds(block_idx * block_t, block_t), r, :],
                    buf.at[slot, :, r, :],
                    sem.at[slot]).start()

    @pl.when(b == 0)
    def _():
        for i in range(n_slots - 1):
            issue(i, i)
        for r in range(n_routes):
            @pl.when(jnp.logical_not(keeps[r]))
            def _():
                for s in range(n_slots):
                    buf[s, :, r, :] = jnp.zeros((block_t, buf.shape[3]),
                                                jnp.float32)

    @pl.when(b + n_slots - 1 < nb)
    def _():
        issue(b + n_slots - 1, (b + n_slots - 1) % n_slots)

    cur = b % n_slots
    for r in range(n_routes):
        @pl.when(keeps[r])
        def _():
            pltpu.make_async_copy(
                x_hbm.at[pl.ds(0, block_t), r, :],
                buf.at[cur, :, r, :],
                sem.at[cur]).wait()
    o_ref[...] = buf[cur] * jnp.float32(scale)


def _sc_mask_body(n_routes, n_keep, imp_hbm, noise_hbm, km_hbm,
                  imp_v, noise_v, score_v, km_v):
    c = lax.axis_index("c")
    s = lax.axis_index("s")

    @pl.when(jnp.logical_and(c == 0, s == 0))
    def _():
        pltpu.sync_copy(imp_hbm, imp_v)
        pltpu.sync_copy(noise_hbm, noise_v)
        score = 1.0 / (imp_v[...] + _EPS) + noise_v[...]
        score_v[...] = score
        iota = lax.iota(jnp.int32, n_routes)
        rank = jnp.zeros((n_routes,), jnp.int32)
        neg_inf = jnp.float32(-3.4e38)
        for j in range(n_routes):
            sj_scalar = lax.reduce_max(
                jnp.where(iota == j, score, neg_inf), (0,))
            sj = lax.broadcast_in_dim(sj_scalar, (n_routes,), ())
            beats = jnp.logical_or(
                sj < score,
                jnp.logical_and(sj == score,
                                jnp.full((n_routes,), j, jnp.int32) < iota))
            rank = rank + jnp.where(beats, jnp.int32(1), jnp.int32(0))
        km_v[...] = jnp.where(rank < n_keep, jnp.float32(1.0),
                              jnp.float32(0.0))
        pltpu.sync_copy(km_v, km_hbm)


def _sc_keep_mask(importance, noise, n_routes, n_keep):
    mesh = plsc.VectorSubcoreMesh(core_axis_name="c", subcore_axis_name="s")
    kern = functools.partial(
        pl.kernel,
        mesh=mesh,
        out_type=jax.ShapeDtypeStruct((n_routes,), jnp.float32),
        scratch_types=[pltpu.VMEM((n_routes,), jnp.float32),
                       pltpu.VMEM((n_routes,), jnp.float32),
                       pltpu.VMEM((n_routes,), jnp.float32),
                       pltpu.VMEM((n_routes,), jnp.float32)],
        compiler_params=pltpu.CompilerParams(needs_layout_passes=False),
    )(functools.partial(_sc_mask_body, n_routes, n_keep))
    return kern(importance, noise)


def kernel(x, importance):
    n_tokens, n_routes, d = x.shape
    n_keep = max(_MIN_KEEP, int(n_routes * (1.0 - _DROP_PROB)))
    scale = n_routes / float(n_keep)
    noise = jax.random.uniform(jax.random.key(42), (n_routes,),
                               importance.dtype) * 0.5

    block_t = 128
    nb = n_tokens // block_t
    body = functools.partial(_body, n_routes=n_routes, n_keep=n_keep,
                             scale=scale, block_t=block_t, nb=nb)
    out, km = pl.pallas_call(
        body,
        grid=(nb,),
        in_specs=[pl.BlockSpec(memory_space=pltpu.SMEM),
                  pl.BlockSpec(memory_space=pltpu.SMEM),
                  pl.BlockSpec(memory_space=pl.ANY)],
        out_specs=[pl.BlockSpec((block_t, n_routes, d), lambda b: (b, 0, 0)),
                   pl.BlockSpec(memory_space=pltpu.SMEM)],
        out_shape=[jax.ShapeDtypeStruct((n_tokens, n_routes, d), jnp.float32),
                   jax.ShapeDtypeStruct((n_routes,), jnp.float32)],
        scratch_shapes=[pltpu.VMEM((4, block_t, n_routes, d), jnp.float32),
                        pltpu.SemaphoreType.DMA((4,)),
                        pltpu.SMEM((n_routes,), jnp.int32)],
        compiler_params=pltpu.CompilerParams(
            dimension_semantics=("arbitrary",)),
    )(importance, noise, x)
    return out, km
